# baseline (speedup 1.0000x reference)
import functools

import jax
import jax.numpy as jnp
from jax import lax
from jax.experimental import pallas as pl
from jax.experimental.pallas import tpu as pltpu
from jax.experimental.pallas import tpu_sc as plsc

B = 16384
D = 128
L = 16
NC = 2
NS = 16
NW = NC * NS
B_W = B // NW
C = 128
NCH = B_W // C
G = C // L


def _mf_body(user_h, item_h, uew_h, iew_h, ubw_h, ibw_h, out_h,
             uidx, iidx, ue, ie, ub, ib, outc, sems, semb, semi):
    cid = lax.axis_index("c")
    sid = lax.axis_index("s")
    wid = sid * NC + cid
    base = wid * B_W

    def start(c, buf):
        return (pltpu.async_copy(uew_h.at[uidx.at[c]], ue.at[buf], sems[buf]),
                pltpu.async_copy(iew_h.at[iidx.at[c]], ie.at[buf], sems[buf]))

    h0u = pltpu.async_copy(user_h.at[wid, 0], uidx.at[0], semi)
    h0i = pltpu.async_copy(item_h.at[wid, 0], iidx.at[0], semi)
    h0u.wait()
    h0i.wait()
    pending = {0: start(0, 0)}
    hru = pltpu.async_copy(user_h.at[wid, pl.ds(1, NCH - 1)],
                           uidx.at[pl.ds(1, NCH - 1)], semi)
    hri = pltpu.async_copy(item_h.at[wid, pl.ds(1, NCH - 1)],
                           iidx.at[pl.ds(1, NCH - 1)], semi)
    hru.wait()
    hri.wait()
    pending[1] = start(1, 1)

    bias_handles = []
    for c in range(NCH):
        bias_handles.append(pltpu.async_copy(ubw_h.at[uidx.at[c]], ub.at[c], semb))
        bias_handles.append(pltpu.async_copy(ibw_h.at[iidx.at[c]], ib.at[c], semb))

    lane = lax.iota(jnp.int32, L)
    rows = [lane + (g * L) for g in range(G)]

    for c in range(NCH):
        buf = c % 2
        if c + 1 < NCH and c + 1 not in pending:
            pending[c + 1] = start(c + 1, 1 - buf)
        for h in pending.pop(c):
            h.wait()

        uec = ue.at[buf]
        iec = ie.at[buf]

        def fbody(f, accs):
            cols = jnp.bitwise_and(lane + f, D - 1)
            out = []
            for g in range(G):
                uv = plsc.load_gather(uec, [rows[g], cols])
                iv = plsc.load_gather(iec, [rows[g], cols])
                out.append(accs[g] + uv * iv)
            return tuple(out)

        accs = lax.fori_loop(
            0, D, fbody, tuple(jnp.zeros((L,), jnp.float32) for _ in range(G)),
            unroll=4,
        )
        if c == 0:
            for h in bias_handles:
                h.wait()
        for g in range(G):
            res = accs[g] + ub[c, pl.ds(g * L, L)] + ib[c, pl.ds(g * L, L)]
            outc[pl.ds(g * L, L)] = res

        off = pl.multiple_of(base + c * C, C)
        pltpu.sync_copy(outc, out_h.at[pl.ds(off, C)])


_mf = functools.partial(
    pl.kernel,
    out_type=jax.ShapeDtypeStruct((B,), jnp.float32),
    mesh=plsc.VectorSubcoreMesh(core_axis_name="c", subcore_axis_name="s"),
    compiler_params=pltpu.CompilerParams(needs_layout_passes=False),
    scratch_types=[
        pltpu.VMEM((NCH, C), jnp.int32),
        pltpu.VMEM((NCH, C), jnp.int32),
        pltpu.VMEM((2, C, D), jnp.float32),
        pltpu.VMEM((2, C, D), jnp.float32),
        pltpu.VMEM((NCH, C), jnp.float32),
        pltpu.VMEM((NCH, C), jnp.float32),
        pltpu.VMEM((C,), jnp.float32),
        [pltpu.SemaphoreType.DMA, pltpu.SemaphoreType.DMA],
        pltpu.SemaphoreType.DMA,
        pltpu.SemaphoreType.DMA,
    ],
)(_mf_body)


@jax.jit
def kernel(user, item, user_embed_w, item_embed_w, user_bias_w, item_bias_w):
    user_r = user.astype(jnp.int32).reshape(NW, NCH, C)
    item_r = item.astype(jnp.int32).reshape(NW, NCH, C)
    ub_flat = user_bias_w.reshape(-1)
    ib_flat = item_bias_w.reshape(-1)
    return _mf(user_r, item_r, user_embed_w, item_embed_w, ub_flat, ib_flat)


# speedup vs baseline: 1.0372x; 1.0372x over previous
import functools

import jax
import jax.numpy as jnp
from jax import lax
from jax.experimental import pallas as pl
from jax.experimental.pallas import tpu as pltpu
from jax.experimental.pallas import tpu_sc as plsc

B = 16384
D = 128
L = 16
NC = 2
NS = 16
NW = NC * NS
B_W = B // NW
C = 128
NCH = B_W // C
G = C // L


def _mf_body(user_h, item_h, uew_h, iew_h, ubw_h, ibw_h, out_h,
             uidx, iidx, ue, ie, ub, ib, outc, sems, semb, semi):
    cid = lax.axis_index("c")
    sid = lax.axis_index("s")
    wid = sid * NC + cid
    base = wid * B_W

    def start(c, buf):
        return (pltpu.async_copy(uew_h.at[uidx.at[c]], ue.at[buf], sems[buf]),
                pltpu.async_copy(iew_h.at[iidx.at[c]], ie.at[buf], sems[buf]))

    h0u = pltpu.async_copy(user_h.at[wid, 0], uidx.at[0], semi)
    h0i = pltpu.async_copy(item_h.at[wid, 0], iidx.at[0], semi)
    h0u.wait()
    h0i.wait()
    pending = {0: start(0, 0)}
    hru = pltpu.async_copy(user_h.at[wid, pl.ds(1, NCH - 1)],
                           uidx.at[pl.ds(1, NCH - 1)], semi)
    hri = pltpu.async_copy(item_h.at[wid, pl.ds(1, NCH - 1)],
                           iidx.at[pl.ds(1, NCH - 1)], semi)
    hru.wait()
    hri.wait()
    pending[1] = start(1, 1)

    bias_handles = []
    for c in range(NCH):
        bias_handles.append(pltpu.async_copy(ubw_h.at[uidx.at[c]], ub.at[c], semb))
        bias_handles.append(pltpu.async_copy(ibw_h.at[iidx.at[c]], ib.at[c], semb))

    lane = lax.iota(jnp.int32, L)
    rows = [lane + (g * L) for g in range(G)]

    for c in range(NCH):
        buf = c % 2
        if c + 1 < NCH and c + 1 not in pending:
            pending[c + 1] = start(c + 1, 1 - buf)
        for h in pending.pop(c):
            h.wait()

        uec = ue.at[buf]
        iec = ie.at[buf]

        def fbody(f, accs):
            cols = jnp.bitwise_and(lane + f, D - 1)
            out = []
            for g in range(G):
                uv = plsc.load_gather(uec, [rows[g], cols])
                iv = plsc.load_gather(iec, [rows[g], cols])
                out.append(accs[g] + uv * iv)
            return tuple(out)

        accs = lax.fori_loop(
            0, D, fbody, tuple(jnp.zeros((L,), jnp.float32) for _ in range(G)),
            unroll=2,
        )
        if c == 0:
            for h in bias_handles:
                h.wait()
        for g in range(G):
            res = accs[g] + ub[c, pl.ds(g * L, L)] + ib[c, pl.ds(g * L, L)]
            outc[pl.ds(g * L, L)] = res

        off = pl.multiple_of(base + c * C, C)
        pltpu.sync_copy(outc, out_h.at[pl.ds(off, C)])


_mf = functools.partial(
    pl.kernel,
    out_type=jax.ShapeDtypeStruct((B,), jnp.float32),
    mesh=plsc.VectorSubcoreMesh(core_axis_name="c", subcore_axis_name="s"),
    compiler_params=pltpu.CompilerParams(needs_layout_passes=False),
    scratch_types=[
        pltpu.VMEM((NCH, C), jnp.int32),
        pltpu.VMEM((NCH, C), jnp.int32),
        pltpu.VMEM((2, C, D), jnp.float32),
        pltpu.VMEM((2, C, D), jnp.float32),
        pltpu.VMEM((NCH, C), jnp.float32),
        pltpu.VMEM((NCH, C), jnp.float32),
        pltpu.VMEM((C,), jnp.float32),
        [pltpu.SemaphoreType.DMA, pltpu.SemaphoreType.DMA],
        pltpu.SemaphoreType.DMA,
        pltpu.SemaphoreType.DMA,
    ],
)(_mf_body)


@jax.jit
def kernel(user, item, user_embed_w, item_embed_w, user_bias_w, item_bias_w):
    user_r = user.astype(jnp.int32).reshape(NW, NCH, C)
    item_r = item.astype(jnp.int32).reshape(NW, NCH, C)
    ub_flat = user_bias_w.reshape(-1)
    ib_flat = item_bias_w.reshape(-1)
    return _mf(user_r, item_r, user_embed_w, item_embed_w, ub_flat, ib_flat)
